# baseline (speedup 1.0000x reference)
import functools

import jax
import jax.numpy as jnp
from jax import lax
from jax.experimental import pallas as pl
from jax.experimental.pallas import tpu as pltpu
from jax.experimental.pallas import tpu_sc as plsc

_N_EMBD = 768
_N_EXP = 64
_E_DIM = 192
_N_TOK = 8192
_TB = 256
_B = 128
_NB_MAX = _N_TOK // _B + _N_EXP
_PAD = _NB_MAX * _B

_NC = 2
_NS = 16
_NW = _NC * _NS
_CHUNK = _N_TOK // _NW
_NSUB = 4
_SUB = _CHUNK // _NSUB


def _router_body(x_ref, wr_ref, eid_ref, rank_ref, counts_ref, run_ref):
    i = pl.program_id(0)

    @pl.when(i == 0)
    def _():
        run_ref[...] = jnp.zeros_like(run_ref)

    xb = x_ref[...]
    logits = jnp.dot(xb, wr_ref[...], preferred_element_type=jnp.float32)
    m = jnp.max(logits, axis=1, keepdims=True)
    col = lax.broadcasted_iota(jnp.int32, logits.shape, 1)
    eid = jnp.min(jnp.where(logits == m, col, _N_EXP), axis=1)
    eid_ref[0, 0, :] = eid

    onehot = (col == eid[:, None]).astype(jnp.float32)
    r = lax.broadcasted_iota(jnp.int32, (_TB, _TB), 0)
    c = lax.broadcasted_iota(jnp.int32, (_TB, _TB), 1)
    tril = (r >= c).astype(jnp.float32)
    prefix = jnp.dot(tril, onehot, preferred_element_type=jnp.float32)
    rank_in_blk = jnp.sum(prefix * onehot, axis=1) - 1.0
    run = run_ref[...]
    rank = rank_in_blk + jnp.sum(run * onehot, axis=1)
    rank_ref[0, 0, :] = rank.astype(jnp.int32)
    run = run + jnp.sum(onehot, axis=0, keepdims=True)
    run_ref[...] = run
    counts_ref[...] = jnp.broadcast_to(run, (8, _N_EXP))


def _router(x, Wr):
    nb = _N_TOK // _TB
    return pl.pallas_call(
        _router_body,
        grid=(nb,),
        in_specs=[
            pl.BlockSpec((_TB, _N_EMBD), lambda i: (i, 0)),
            pl.BlockSpec((_N_EMBD, _N_EXP), lambda i: (0, 0)),
        ],
        out_specs=[
            pl.BlockSpec((1, 1, _TB), lambda i: (i // 4, 0, i % 4)),
            pl.BlockSpec((1, 1, _TB), lambda i: (i // 4, 0, i % 4)),
            pl.BlockSpec((8, _N_EXP), lambda i: (0, 0)),
        ],
        out_shape=[
            jax.ShapeDtypeStruct((_N_TOK // _DTB, 1, _DTB), jnp.int32),
            jax.ShapeDtypeStruct((_N_TOK // _DTB, 1, _DTB), jnp.int32),
            jax.ShapeDtypeStruct((8, _N_EXP), jnp.float32),
        ],
        scratch_shapes=[pltpu.VMEM((1, _N_EXP), jnp.float32)],
    )(x, Wr)


_DTB = 1024


def _dest_body(eid_ref, rank_ref, tsr_ref, cumt_ref, dest_ref, be_ref,
               act_ref, obi_ref):
    eid = eid_ref[0, 0, :]
    onehot = lax.broadcasted_iota(jnp.int32, (_DTB, _N_EXP), 1) == eid[:, None]
    tsr_b = tsr_ref[...][0:1, :]
    sel = jnp.sum(jnp.where(onehot, tsr_b, 0), axis=1)
    dest_ref[0, 0, :] = sel + rank_ref[0, 0, :]

    @pl.when(pl.program_id(0) == 0)
    def _():
        cumt = cumt_ref[...][0:1, :]
        b = lax.broadcasted_iota(jnp.int32, (_NB_MAX, _N_EXP), 0)
        be = jnp.sum((cumt <= b).astype(jnp.int32), axis=1)
        be_ref[0, 0, :] = jnp.minimum(be, _N_EXP - 1)
        total = jnp.max(cumt, axis=1)
        blk = lax.broadcasted_iota(jnp.int32, (_NB_MAX,), 0)
        act_ref[0, 0, :] = (blk < total).astype(jnp.int32)
        obi_ref[0, 0, :] = jnp.minimum(blk, total - 1)


def _dest_kernel(eid3, rank3, tsr8, cumt8):
    nb = _N_TOK // _DTB
    return pl.pallas_call(
        _dest_body,
        grid=(nb,),
        in_specs=[
            pl.BlockSpec((1, 1, _DTB), lambda i: (i, 0, 0)),
            pl.BlockSpec((1, 1, _DTB), lambda i: (i, 0, 0)),
            pl.BlockSpec((8, _N_EXP), lambda i: (0, 0)),
            pl.BlockSpec((8, _N_EXP), lambda i: (0, 0)),
        ],
        out_specs=[
            pl.BlockSpec((1, 1, _DTB), lambda i: (i, 0, 0)),
            pl.BlockSpec((1, 1, _NB_MAX), lambda i: (0, 0, 0)),
            pl.BlockSpec((1, 1, _NB_MAX), lambda i: (0, 0, 0)),
            pl.BlockSpec((1, 1, _NB_MAX), lambda i: (0, 0, 0)),
        ],
        out_shape=[
            jax.ShapeDtypeStruct((nb, 1, _DTB), jnp.int32),
            jax.ShapeDtypeStruct((1, 1, _NB_MAX), jnp.int32),
            jax.ShapeDtypeStruct((1, 1, _NB_MAX), jnp.int32),
            jax.ShapeDtypeStruct((1, 1, _NB_MAX), jnp.int32),
        ],
    )(eid3, rank3, tsr8, cumt8)


@functools.cache
def _make_sc_kernels():
    sc_mesh = plsc.VectorSubcoreMesh(
        core_axis_name="c", subcore_axis_name="s",
        num_cores=_NC, num_subcores=_NS)

    @functools.partial(
        pl.kernel,
        out_type=jax.ShapeDtypeStruct((_PAD, _N_EMBD), jnp.float32),
        mesh=sc_mesh,
        scratch_types=[
            pltpu.VMEM((_NSUB, _SUB), jnp.int32),
            pltpu.VMEM((2, _SUB, _N_EMBD), jnp.float32),
            pltpu.SemaphoreType.DMA,
            pltpu.SemaphoreType.DMA,
        ],
    )
    def sc_dispatch(x_hbm, dest_hbm, xs_pad_hbm, dest_v, rows_v, ldsem, sem):
        wid = lax.axis_index("s") * _NC + lax.axis_index("c")
        base = wid * _CHUNK
        pltpu.sync_copy(dest_hbm.at[wid], dest_v)

        def load(j):
            return pltpu.async_copy(
                x_hbm.at[pl.ds(base + j * _SUB, _SUB)], rows_v.at[j % 2],
                ldsem)

        def scatter(j):
            return pltpu.async_copy(
                rows_v.at[j % 2], xs_pad_hbm.at[dest_v.at[j]], sem)

        ld = {0: load(0), 1: load(1)}
        st = {}
        ld[0].wait(); st[0] = scatter(0)
        ld[1].wait(); st[1] = scatter(1)
        st[0].wait(); ld[2] = load(2)
        st[1].wait(); ld[3] = load(3)
        ld[2].wait(); st[2] = scatter(2)
        ld[3].wait(); st[3] = scatter(3)
        st[2].wait(); st[3].wait()

    @functools.partial(
        pl.kernel,
        out_type=jax.ShapeDtypeStruct((_N_TOK, _N_EMBD), jnp.float32),
        mesh=sc_mesh,
        scratch_types=[
            pltpu.VMEM((_NSUB, _SUB), jnp.int32),
            pltpu.VMEM((2, _SUB, _N_EMBD), jnp.float32),
            pltpu.SemaphoreType.DMA,
            pltpu.SemaphoreType.DMA,
        ],
    )
    def sc_undispatch(out_pad_hbm, dest_hbm, out_hbm, dest_v, rows_v, gsem,
                      stsem):
        wid = lax.axis_index("s") * _NC + lax.axis_index("c")
        base = wid * _CHUNK
        pltpu.sync_copy(dest_hbm.at[wid], dest_v)

        def gather(j):
            return pltpu.async_copy(
                out_pad_hbm.at[dest_v.at[j]], rows_v.at[j % 2], gsem)

        def store(j):
            return pltpu.async_copy(
                rows_v.at[j % 2], out_hbm.at[pl.ds(base + j * _SUB, _SUB)],
                stsem)

        g = {0: gather(0), 1: gather(1)}
        s = {}
        g[0].wait(); s[0] = store(0)
        g[1].wait(); s[1] = store(1)
        s[0].wait(); g[2] = gather(2)
        s[1].wait(); g[3] = gather(3)
        g[2].wait(); s[2] = store(2)
        g[3].wait(); s[3] = store(3)
        s[2].wait(); s[3].wait()

    return sc_dispatch, sc_undispatch


def _sc_dispatch(xbf_i32, dest):
    return _make_sc_kernels()[0](xbf_i32, dest)


def _sc_undispatch(out_pad, dest):
    return _make_sc_kernels()[1](out_pad, dest)


def _ffn_body(be_ref, act_ref, obi_ref, xs_ref, w1s_ref, w2s_ref, w1e_ref,
              w2e_ref, out_ref):
    b = pl.program_id(0)

    @pl.when(act_ref[b] != 0)
    def _():
        xb = xs_ref[...].astype(jnp.bfloat16)
        w1s = w1s_ref[...]
        w2s = w2s_ref[...]
        w1e = w1e_ref[...]
        w2e = w2e_ref[...]
        hs = jnp.dot(xb, w1s, preferred_element_type=jnp.float32)
        hs = jnp.square(jnp.maximum(hs, 0.0)).astype(jnp.bfloat16)
        acc = jnp.dot(hs, w2s, preferred_element_type=jnp.float32)
        he = jnp.dot(xb, w1e, preferred_element_type=jnp.float32)
        he = jnp.square(jnp.maximum(he, 0.0)).astype(jnp.bfloat16)
        acc = acc + jnp.dot(he, w2e, preferred_element_type=jnp.float32)
        out_ref[...] = acc


def _grouped_ffn(xs_pad, W1s, W2s, W1e2, W2e2, block_expert, block_active,
                 out_block):
    grid_spec = pltpu.PrefetchScalarGridSpec(
        num_scalar_prefetch=3,
        grid=(_NB_MAX,),
        in_specs=[
            pl.BlockSpec((_B, _N_EMBD),
                         lambda b, be, act, obi: (act[b] * b, 0)),
            pl.BlockSpec((_N_EMBD, _E_DIM), lambda b, be, act, obi: (0, 0)),
            pl.BlockSpec((_E_DIM, _N_EMBD), lambda b, be, act, obi: (0, 0)),
            pl.BlockSpec((_N_EMBD, _E_DIM),
                         lambda b, be, act, obi: (be[b], 0)),
            pl.BlockSpec((_E_DIM, _N_EMBD),
                         lambda b, be, act, obi: (be[b], 0)),
        ],
        out_specs=pl.BlockSpec((_B, _N_EMBD),
                               lambda b, be, act, obi: (obi[b], 0)),
    )
    return pl.pallas_call(
        _ffn_body,
        grid_spec=grid_spec,
        out_shape=jax.ShapeDtypeStruct((_PAD, _N_EMBD), jnp.float32),
    )(block_expert, block_active, out_block, xs_pad, W1s, W2s, W1e2, W2e2)


@jax.jit
def kernel(x, W1s, W2s, Wr, W1e, W2e):
    x_flat = x.reshape(-1, _N_EMBD)
    eid3, rank3, counts8 = _router(x_flat, Wr)
    counts = counts8[0].astype(jnp.int32)
    tiles = (counts + _B - 1) // _B
    cum_tiles = jnp.cumsum(tiles).astype(jnp.int32)
    tile_start_rows = ((cum_tiles - tiles) * _B).astype(jnp.int32)
    tsr8 = jnp.broadcast_to(tile_start_rows[None, :], (8, _N_EXP))
    cumt8 = jnp.broadcast_to(cum_tiles[None, :], (8, _N_EXP))

    dest3, be3, act3, obi3 = _dest_kernel(eid3, rank3, tsr8, cumt8)
    dest = dest3.reshape(_NW, _NSUB, _SUB)
    be = be3.reshape(_NB_MAX)
    active = act3.reshape(_NB_MAX)
    obi = obi3.reshape(_NB_MAX)
    xs_pad = _sc_dispatch(x_flat, dest)

    out_pad = _grouped_ffn(
        xs_pad,
        W1s.astype(jnp.bfloat16), W2s.astype(jnp.bfloat16),
        W1e.astype(jnp.bfloat16).reshape(_N_EXP * _N_EMBD, _E_DIM),
        W2e.astype(jnp.bfloat16).reshape(_N_EXP * _E_DIM, _N_EMBD),
        be, active, obi)
    out = _sc_undispatch(out_pad, dest)
    return out.reshape(x.shape)


# speedup vs baseline: 1.1215x; 1.1215x over previous
import functools

import jax
import jax.numpy as jnp
from jax import lax
from jax.experimental import pallas as pl
from jax.experimental.pallas import tpu as pltpu
from jax.experimental.pallas import tpu_sc as plsc

_N_EMBD = 768
_N_EXP = 64
_E_DIM = 192
_N_TOK = 8192
_TB = 512
_B = 128
_NB_MAX = _N_TOK // _B + _N_EXP
_PAD = _NB_MAX * _B

_NC = 2
_NS = 16
_NW = _NC * _NS
_CHUNK = _N_TOK // _NW
_NSUB = 4
_SUB = _CHUNK // _NSUB


def _router_body(x_ref, wr_ref, eid_ref, rank_ref, counts_ref, run_ref):
    i = pl.program_id(0)

    @pl.when(i == 0)
    def _():
        run_ref[...] = jnp.zeros_like(run_ref)

    xb = x_ref[...]
    logits = jnp.dot(xb, wr_ref[...], preferred_element_type=jnp.float32)
    m = jnp.max(logits, axis=1, keepdims=True)
    col = lax.broadcasted_iota(jnp.int32, logits.shape, 1)
    eid = jnp.min(jnp.where(logits == m, col, _N_EXP), axis=1)
    eid_ref[0, 0, :] = eid

    onehot = (col == eid[:, None]).astype(jnp.float32)
    r = lax.broadcasted_iota(jnp.int32, (_TB, _TB), 0)
    c = lax.broadcasted_iota(jnp.int32, (_TB, _TB), 1)
    tril = (r >= c).astype(jnp.float32)
    prefix = jnp.dot(tril, onehot, preferred_element_type=jnp.float32)
    rank_in_blk = jnp.sum(prefix * onehot, axis=1) - 1.0
    run = run_ref[...]
    rank = rank_in_blk + jnp.sum(run * onehot, axis=1)
    rank_ref[0, 0, :] = rank.astype(jnp.int32)
    run = run + jnp.sum(onehot, axis=0, keepdims=True)
    run_ref[...] = run
    counts_ref[...] = jnp.broadcast_to(run, (8, _N_EXP))


def _router(x, Wr):
    nb = _N_TOK // _TB
    return pl.pallas_call(
        _router_body,
        grid=(nb,),
        in_specs=[
            pl.BlockSpec((_TB, _N_EMBD), lambda i: (i, 0)),
            pl.BlockSpec((_N_EMBD, _N_EXP), lambda i: (0, 0)),
        ],
        out_specs=[
            pl.BlockSpec((1, 1, _TB), lambda i: (i // 2, 0, i % 2)),
            pl.BlockSpec((1, 1, _TB), lambda i: (i // 2, 0, i % 2)),
            pl.BlockSpec((8, _N_EXP), lambda i: (0, 0)),
        ],
        out_shape=[
            jax.ShapeDtypeStruct((_N_TOK // _DTB, 1, _DTB), jnp.int32),
            jax.ShapeDtypeStruct((_N_TOK // _DTB, 1, _DTB), jnp.int32),
            jax.ShapeDtypeStruct((8, _N_EXP), jnp.float32),
        ],
        scratch_shapes=[pltpu.VMEM((1, _N_EXP), jnp.float32)],
    )(x, Wr)


_DTB = 1024


def _dest_body(eid_ref, rank_ref, tsr_ref, cumt_ref, dest_ref, be_ref,
               act_ref, obi_ref):
    eid = eid_ref[0, 0, :]
    onehot = lax.broadcasted_iota(jnp.int32, (_DTB, _N_EXP), 1) == eid[:, None]
    tsr_b = tsr_ref[...][0:1, :]
    sel = jnp.sum(jnp.where(onehot, tsr_b, 0), axis=1)
    dest_ref[0, 0, :] = sel + rank_ref[0, 0, :]

    @pl.when(pl.program_id(0) == 0)
    def _():
        cumt = cumt_ref[...][0:1, :]
        b = lax.broadcasted_iota(jnp.int32, (_NB_MAX, _N_EXP), 0)
        be = jnp.sum((cumt <= b).astype(jnp.int32), axis=1)
        be_ref[0, 0, :] = jnp.minimum(be, _N_EXP - 1)
        total = jnp.max(cumt, axis=1)
        blk = lax.broadcasted_iota(jnp.int32, (_NB_MAX,), 0)
        act_ref[0, 0, :] = (blk < total).astype(jnp.int32)
        obi_ref[0, 0, :] = jnp.minimum(blk, total - 1)


def _dest_kernel(eid3, rank3, tsr8, cumt8):
    nb = _N_TOK // _DTB
    return pl.pallas_call(
        _dest_body,
        grid=(nb,),
        in_specs=[
            pl.BlockSpec((1, 1, _DTB), lambda i: (i, 0, 0)),
            pl.BlockSpec((1, 1, _DTB), lambda i: (i, 0, 0)),
            pl.BlockSpec((8, _N_EXP), lambda i: (0, 0)),
            pl.BlockSpec((8, _N_EXP), lambda i: (0, 0)),
        ],
        out_specs=[
            pl.BlockSpec((1, 1, _DTB), lambda i: (i, 0, 0)),
            pl.BlockSpec((1, 1, _NB_MAX), lambda i: (0, 0, 0)),
            pl.BlockSpec((1, 1, _NB_MAX), lambda i: (0, 0, 0)),
            pl.BlockSpec((1, 1, _NB_MAX), lambda i: (0, 0, 0)),
        ],
        out_shape=[
            jax.ShapeDtypeStruct((nb, 1, _DTB), jnp.int32),
            jax.ShapeDtypeStruct((1, 1, _NB_MAX), jnp.int32),
            jax.ShapeDtypeStruct((1, 1, _NB_MAX), jnp.int32),
            jax.ShapeDtypeStruct((1, 1, _NB_MAX), jnp.int32),
        ],
    )(eid3, rank3, tsr8, cumt8)


@functools.cache
def _make_sc_kernels():
    sc_mesh = plsc.VectorSubcoreMesh(
        core_axis_name="c", subcore_axis_name="s",
        num_cores=_NC, num_subcores=_NS)

    @functools.partial(
        pl.kernel,
        out_type=jax.ShapeDtypeStruct((_PAD, _N_EMBD), jnp.float32),
        mesh=sc_mesh,
        scratch_types=[
            pltpu.VMEM((_NSUB, _SUB), jnp.int32),
            pltpu.VMEM((2, _SUB, _N_EMBD), jnp.float32),
            pltpu.SemaphoreType.DMA,
            pltpu.SemaphoreType.DMA,
        ],
    )
    def sc_dispatch(x_hbm, dest_hbm, xs_pad_hbm, dest_v, rows_v, ldsem, sem):
        wid = lax.axis_index("s") * _NC + lax.axis_index("c")
        base = wid * _CHUNK
        pltpu.sync_copy(dest_hbm.at[wid], dest_v)

        def load(j):
            return pltpu.async_copy(
                x_hbm.at[pl.ds(base + j * _SUB, _SUB)], rows_v.at[j % 2],
                ldsem)

        def scatter(j):
            return pltpu.async_copy(
                rows_v.at[j % 2], xs_pad_hbm.at[dest_v.at[j]], sem)

        ld = {0: load(0), 1: load(1)}
        st = {}
        ld[0].wait(); st[0] = scatter(0)
        ld[1].wait(); st[1] = scatter(1)
        st[0].wait(); ld[2] = load(2)
        st[1].wait(); ld[3] = load(3)
        ld[2].wait(); st[2] = scatter(2)
        ld[3].wait(); st[3] = scatter(3)
        st[2].wait(); st[3].wait()

    @functools.partial(
        pl.kernel,
        out_type=jax.ShapeDtypeStruct((_N_TOK, _N_EMBD), jnp.float32),
        mesh=sc_mesh,
        scratch_types=[
            pltpu.VMEM((_NSUB, _SUB), jnp.int32),
            pltpu.VMEM((2, _SUB, _N_EMBD), jnp.float32),
            pltpu.SemaphoreType.DMA,
            pltpu.SemaphoreType.DMA,
        ],
    )
    def sc_undispatch(out_pad_hbm, dest_hbm, out_hbm, dest_v, rows_v, gsem,
                      stsem):
        wid = lax.axis_index("s") * _NC + lax.axis_index("c")
        base = wid * _CHUNK
        pltpu.sync_copy(dest_hbm.at[wid], dest_v)

        def gather(j):
            return pltpu.async_copy(
                out_pad_hbm.at[dest_v.at[j]], rows_v.at[j % 2], gsem)

        def store(j):
            return pltpu.async_copy(
                rows_v.at[j % 2], out_hbm.at[pl.ds(base + j * _SUB, _SUB)],
                stsem)

        g = {0: gather(0), 1: gather(1)}
        s = {}
        g[0].wait(); s[0] = store(0)
        g[1].wait(); s[1] = store(1)
        s[0].wait(); g[2] = gather(2)
        s[1].wait(); g[3] = gather(3)
        g[2].wait(); s[2] = store(2)
        g[3].wait(); s[3] = store(3)
        s[2].wait(); s[3].wait()

    return sc_dispatch, sc_undispatch


def _sc_dispatch(xbf_i32, dest):
    return _make_sc_kernels()[0](xbf_i32, dest)


def _sc_undispatch(out_pad, dest):
    return _make_sc_kernels()[1](out_pad, dest)


def _ffn_body(be_ref, act_ref, obi_ref, xs_ref, w1s_ref, w2s_ref, w1e_ref,
              w2e_ref, out_ref):
    b = pl.program_id(0)

    @pl.when(act_ref[b] != 0)
    def _():
        xb = xs_ref[...].astype(jnp.bfloat16)
        w1s = w1s_ref[...].astype(jnp.bfloat16)
        w2s = w2s_ref[...].astype(jnp.bfloat16)
        w1e = w1e_ref[...].astype(jnp.bfloat16)
        w2e = w2e_ref[...].astype(jnp.bfloat16)
        hs = jnp.dot(xb, w1s, preferred_element_type=jnp.float32)
        hs = jnp.square(jnp.maximum(hs, 0.0)).astype(jnp.bfloat16)
        acc = jnp.dot(hs, w2s, preferred_element_type=jnp.float32)
        he = jnp.dot(xb, w1e, preferred_element_type=jnp.float32)
        he = jnp.square(jnp.maximum(he, 0.0)).astype(jnp.bfloat16)
        acc = acc + jnp.dot(he, w2e, preferred_element_type=jnp.float32)
        out_ref[...] = acc


def _grouped_ffn(xs_pad, W1s, W2s, W1e2, W2e2, block_expert, block_active,
                 out_block):
    grid_spec = pltpu.PrefetchScalarGridSpec(
        num_scalar_prefetch=3,
        grid=(_NB_MAX,),
        in_specs=[
            pl.BlockSpec((_B, _N_EMBD),
                         lambda b, be, act, obi: (act[b] * b, 0)),
            pl.BlockSpec((_N_EMBD, _E_DIM), lambda b, be, act, obi: (0, 0)),
            pl.BlockSpec((_E_DIM, _N_EMBD), lambda b, be, act, obi: (0, 0)),
            pl.BlockSpec((_N_EMBD, _E_DIM),
                         lambda b, be, act, obi: (be[b], 0)),
            pl.BlockSpec((_E_DIM, _N_EMBD),
                         lambda b, be, act, obi: (be[b], 0)),
        ],
        out_specs=pl.BlockSpec((_B, _N_EMBD),
                               lambda b, be, act, obi: (obi[b], 0)),
    )
    return pl.pallas_call(
        _ffn_body,
        grid_spec=grid_spec,
        out_shape=jax.ShapeDtypeStruct((_PAD, _N_EMBD), jnp.float32),
    )(block_expert, block_active, out_block, xs_pad, W1s, W2s, W1e2, W2e2)


@jax.jit
def kernel(x, W1s, W2s, Wr, W1e, W2e):
    x_flat = x.reshape(-1, _N_EMBD)
    eid3, rank3, counts8 = _router(x_flat, Wr)
    counts = counts8[0].astype(jnp.int32)
    tiles = (counts + _B - 1) // _B
    cum_tiles = jnp.cumsum(tiles).astype(jnp.int32)
    tile_start_rows = ((cum_tiles - tiles) * _B).astype(jnp.int32)
    tsr8 = jnp.broadcast_to(tile_start_rows[None, :], (8, _N_EXP))
    cumt8 = jnp.broadcast_to(cum_tiles[None, :], (8, _N_EXP))

    dest3, be3, act3, obi3 = _dest_kernel(eid3, rank3, tsr8, cumt8)
    dest = dest3.reshape(_NW, _NSUB, _SUB)
    be = be3.reshape(_NB_MAX)
    active = act3.reshape(_NB_MAX)
    obi = obi3.reshape(_NB_MAX)
    xs_pad = _sc_dispatch(x_flat, dest)

    out_pad = _grouped_ffn(
        xs_pad, W1s, W2s,
        W1e.reshape(_N_EXP * _N_EMBD, _E_DIM),
        W2e.reshape(_N_EXP * _E_DIM, _N_EMBD),
        be, active, obi)
    out = _sc_undispatch(out_pad, dest)
    return out.reshape(x.shape)
